# Initial kernel scaffold; baseline (speedup 1.0000x reference)
import jax
import jax.numpy as jnp
from jax.experimental import pallas as pl


def kernel(x, plane_xy, plane_xz, plane_yz):
    raise NotImplementedError("write your pallas kernel here")


# speedup vs baseline: 48.0183x; 48.0183x over previous
import functools

import jax
import jax.numpy as jnp
from jax import lax
from jax.experimental import pallas as pl
from jax.experimental.pallas import tpu as pltpu
from jax.experimental.pallas import tpu_sc as plsc

_C = 32
_R = 512
_N = 1048576
_NC = 2
_NS = 16
_NW = _NC * _NS
_PW = _N // _NW
_B = 128
_NCHUNK = _PW // _B
_L = 16


def _axis_idx(g):
    i = (g + 1.0) * jnp.float32(0.5 * (_R - 1))
    i0 = jnp.minimum(i.astype(jnp.int32), _R - 2)
    w = i - i0.astype(jnp.float32)
    return i0, w


def _tri_body(gx_hbm, gy_hbm, gz_hbm, txy, txz, tyz, out_hbm,
              cx, cy, cz, idx_s, w_s, rows_s, out_v, sem):
    wid = lax.axis_index("s") * _NC + lax.axis_index("c")

    def chunk(ci, carry):
        base = wid * _PW + ci * _B
        pltpu.sync_copy(gx_hbm.at[pl.ds(base, _B)], cx)
        pltpu.sync_copy(gy_hbm.at[pl.ds(base, _B)], cy)
        pltpu.sync_copy(gz_hbm.at[pl.ds(base, _B)], cz)

        for j in range(_B // _L):
            s = pl.ds(j * _L, _L)
            ix0, wx = _axis_idx(cx[s])
            iy0, wy = _axis_idx(cy[s])
            iz0, wz = _axis_idx(cz[s])
            for p, (r0, c0, wr, wc) in enumerate((
                    (iy0, ix0, wy, wx),
                    (iz0, ix0, wz, wx),
                    (iz0, iy0, wz, wy))):
                b00 = r0 * _R + c0
                idx_s[4 * p + 0][s] = b00
                idx_s[4 * p + 1][s] = b00 + 1
                idx_s[4 * p + 2][s] = b00 + _R
                idx_s[4 * p + 3][s] = b00 + _R + 1
                w_s[4 * p + 0][s] = (1.0 - wr) * (1.0 - wc)
                w_s[4 * p + 1][s] = (1.0 - wr) * wc
                w_s[4 * p + 2][s] = wr * (1.0 - wc)
                w_s[4 * p + 3][s] = wr * wc

        tables = (txy, txy, txy, txy, txz, txz, txz, txz, tyz, tyz, tyz, tyz)
        copies = [pltpu.async_copy(tables[k].at[idx_s[k]], rows_s[k], sem)
                  for k in range(12)]
        for c in copies:
            c.wait()

        def comb(j, carry2):
            for p in range(3):
                wv = [w_s[4 * p + k][pl.ds(j * _L, _L)] for k in range(4)]
                r00, r01, r10, r11 = rows_s[4 * p:4 * p + 4]
                for b2 in range(_L):
                    b = j * _L + b2
                    sb = jnp.full((_L,), b2, jnp.int32)
                    dn = lax.GatherDimensionNumbers(
                        offset_dims=(), collapsed_slice_dims=(0,),
                        start_index_map=(0,))
                    w00, w01, w10, w11 = (
                        lax.gather(w, sb[:, None], dn, (1,),
                                   mode=lax.GatherScatterMode.PROMISE_IN_BOUNDS)
                        for w in wv)
                    for h in range(2):
                        f = pl.ds(h * _L, _L)
                        acc = (w00 * r00[b, f] + w01 * r01[b, f]
                               + w10 * r10[b, f] + w11 * r11[b, f])
                        out_v[b, pl.ds(_C * p + h * _L, _L)] = acc
            return carry2

        lax.fori_loop(0, _B // _L, comb, 0)
        pltpu.sync_copy(out_v, out_hbm.at[pl.ds(base, _B)])
        return carry

    lax.fori_loop(0, _NCHUNK, chunk, 0)


@jax.jit
def _tri_sc(gx, gy, gz, txy, txz, tyz):
    scratch = (
        pltpu.VMEM((_B,), jnp.float32),
        pltpu.VMEM((_B,), jnp.float32),
        pltpu.VMEM((_B,), jnp.float32),
        tuple(pltpu.VMEM((_B,), jnp.int32) for _ in range(12)),
        tuple(pltpu.VMEM((_B,), jnp.float32) for _ in range(12)),
        tuple(pltpu.VMEM((_B, _C), jnp.float32) for _ in range(12)),
        pltpu.VMEM((_B, 3 * _C), jnp.float32),
        pltpu.SemaphoreType.DMA,
    )
    kfn = functools.partial(
        pl.kernel,
        out_type=jax.ShapeDtypeStruct((_N, 3 * _C), jnp.float32),
        mesh=plsc.VectorSubcoreMesh(core_axis_name="c", subcore_axis_name="s"),
        scratch_types=scratch,
        compiler_params=pltpu.CompilerParams(use_tc_tiling_on_sc=False),
    )(_tri_body)
    return kfn(gx, gy, gz, txy, txz, tyz)


def kernel(x, plane_xy, plane_xz, plane_yz):
    gx = x[:, 0]
    gy = x[:, 1]
    gz = x[:, 2]
    txy = jnp.transpose(plane_xy[0], (1, 2, 0)).reshape(_R * _R, _C)
    txz = jnp.transpose(plane_xz[0], (1, 2, 0)).reshape(_R * _R, _C)
    tyz = jnp.transpose(plane_yz[0], (1, 2, 0)).reshape(_R * _R, _C)
    return _tri_sc(gx, gy, gz, txy, txz, tyz)
